# baseline (speedup 1.0000x reference)
import functools

import jax
import jax.numpy as jnp
from jax import lax
from jax.experimental import pallas as pl
from jax.experimental.pallas import tpu as pltpu
from jax.experimental.pallas import tpu_sc as plsc

N = 10000
NPAD = 10240
E = 320000
D = 128
CH = 64
NCHUNK = 5120
EPAD = NCHUNK * CH - E
NC = 2
NT = 16
CPT = NCHUNK // (NC * NT)
PH = CPT // 2
ACCR = 10112
RPT = ACCR // NT

_mesh = plsc.VectorSubcoreMesh(core_axis_name="c", subcore_axis_name="s")

_f32 = jnp.float32
_i32 = jnp.int32


EC = 64
EK = 320 // EC


def _embed_body(nf, nt, out, idx, buf0, buf1, sg0, sg1, sw0, sw1):
    bufs = (buf0, buf1)
    sg = (sg0, sg1)
    sw = (sw0, sw1)
    wid = lax.axis_index("s") * NC + lax.axis_index("c")
    wbase = wid * 320
    pltpu.sync_copy(nf.at[pl.ds(wid * EK, EK)], idx)

    def gather(k, b):
        pltpu.async_copy(nt.at[idx.at[k, 0]], bufs[b], sg[b])

    def wb(k, b):
        return pltpu.make_async_copy(
            bufs[b], out.at[pl.ds(wbase + k * EC, EC)], sw[b])

    gather(0, 0)
    for k in range(EK):
        b = k % 2
        nb = 1 - b
        pltpu.make_async_copy(nt.at[idx.at[k, 0]], bufs[b], sg[b]).wait()
        if k + 1 < EK:
            if k >= 1:
                wb(k - 1, nb).wait()
            gather(k + 1, nb)
        pltpu.async_copy(bufs[b], out.at[pl.ds(wbase + k * EC, EC)], sw[b])
    for k in (EK - 2, EK - 1):
        wb(k, k % 2).wait()


def _embed(nf, nt):
    return pl.kernel(
        _embed_body,
        out_type=jax.ShapeDtypeStruct((NPAD, D), _f32),
        mesh=_mesh,
        scratch_types=[
            pltpu.VMEM((EK, 1, EC), _i32),
            pltpu.VMEM((EC, D), _f32),
            pltpu.VMEM((EC, D), _f32),
            pltpu.SemaphoreType.DMA,
            pltpu.SemaphoreType.DMA,
            pltpu.SemaphoreType.DMA,
            pltpu.SemaphoreType.DMA,
        ],
    )(nf, nt)


def _onehot_lookup(idx_col, tab_ref, ncls):
    npd = tab_ref.shape[0]
    oh = (idx_col == lax.broadcasted_iota(_i32, (idx_col.shape[0], npd), 1))
    del ncls
    return jnp.dot(oh.astype(_f32), tab_ref[...],
                   preferred_element_type=_f32)


def _t0_body(g_ref, tf_ref, lf_ref, af_ref, tt_ref, lt_ref, at_ref, o_ref):
    o_ref[...] = jnp.concatenate(
        [g_ref[:, :64],
         _onehot_lookup(tf_ref[...], tt_ref, 20),
         _onehot_lookup(lf_ref[...], lt_ref, 100),
         _onehot_lookup(af_ref[...], at_ref, 10)], axis=1)


def _t0(g, tf2, lf2, af2, tt_p, lt_p, at_p):
    blk = 2560
    return pl.pallas_call(
        _t0_body,
        grid=(NPAD // blk,),
        in_specs=[
            pl.BlockSpec((blk, D), lambda i: (i, 0)),
            pl.BlockSpec((blk, 1), lambda i: (i, 0)),
            pl.BlockSpec((blk, 1), lambda i: (i, 0)),
            pl.BlockSpec((blk, 1), lambda i: (i, 0)),
            pl.BlockSpec((32, 32), lambda i: (0, 0)),
            pl.BlockSpec((128, 16), lambda i: (0, 0)),
            pl.BlockSpec((16, 16), lambda i: (0, 0)),
        ],
        out_specs=pl.BlockSpec((blk, D), lambda i: (i, 0)),
        out_shape=jax.ShapeDtypeStruct((NPAD, D), _f32),
    )(g, tf2, lf2, af2, tt_p, lt_p, at_p)


def _segsum_body(with_hist, x, rows2, cols2, zeros, zrow, *args):
    if with_hist:
        (out, out_h0, out_h1, acc, acc_h, rows_all, cols_all,
         vals0, vals1, ones_v, sg0, sg1, ss0, ss1, sh0, sh1) = args
    else:
        (out, acc, rows_all, cols_all,
         vals0, vals1, sg0, sg1, ss0, ss1) = args
    vals = (vals0, vals1)
    sg = (sg0, sg1)
    ss = (ss0, ss1)
    if with_hist:
        sh = (sh0, sh1)
    c = lax.axis_index("c")
    s = lax.axis_index("s")
    rsl = pl.ds(RPT * s, RPT)
    base = (NT * c + s) * CPT
    pltpu.sync_copy(zeros.at[rsl], acc.at[rsl])
    if with_hist:
        for k in range(CH // 16):
            ones_v[pl.ds(16 * k, 16)] = jnp.full((16,), 1.0, _f32)

        @pl.when(s == 0)
        def _():
            pltpu.sync_copy(zrow.at[pl.ds(0, ACCR)], acc_h)

    plsc.subcore_barrier()

    def gather_start(j, b):
        pltpu.async_copy(x.at[cols_all.at[j]], vals[b], sg[b])

    def gather_wait(j, b):
        pltpu.make_async_copy(x.at[cols_all.at[j]], vals[b], sg[b]).wait()

    def scatter_start(j, b):
        pltpu.async_copy(vals[b], acc.at[rows_all.at[j]], ss[b], add=True)
        if with_hist:
            pltpu.async_copy(ones_v, acc_h.at[rows_all.at[j]], sh[b],
                             add=True)

    def scatter_wait(j, b):
        pltpu.make_async_copy(vals[b], acc.at[rows_all.at[j]], ss[b]).wait()
        if with_hist:
            pltpu.make_async_copy(ones_v, acc_h.at[rows_all.at[j]],
                                  sh[b]).wait()

    def step(ji, carry):
        for u in (0, 1):
            j = 2 * ji + u
            nb = 1 - u
            gather_wait(j, u)
            if u == 0:
                @pl.when(j >= 1)
                def _():
                    scatter_wait(j - 1, nb)

                gather_start(j + 1, nb)
            else:
                scatter_wait(j - 1, nb)

                @pl.when(j + 1 < PH)
                def _():
                    gather_start(j + 1, nb)

            scatter_start(j, u)
        return carry

    for p in range(CPT // PH):
        psl = pl.ds(base + p * PH, PH)
        pltpu.sync_copy(rows2.at[psl], rows_all)
        pltpu.sync_copy(cols2.at[psl], cols_all)
        gather_start(0, 0)
        lax.fori_loop(0, PH // 2, step, 0)
        scatter_wait(PH - 1, 1)
    plsc.subcore_barrier()
    pltpu.sync_copy(acc.at[rsl], out.at[c, rsl])
    if with_hist:
        for oh, cc in ((out_h0, 0), (out_h1, 1)):
            @pl.when((c == cc) & (s < 15))
            def _(oh=oh):
                hsl = pl.ds(640 * s, 640)
                pltpu.sync_copy(acc_h.at[hsl], oh.at[hsl])

            @pl.when((c == cc) & (s == 15))
            def _(oh=oh):
                hsl = pl.ds(9600, 512)
                pltpu.sync_copy(acc_h.at[hsl], oh.at[hsl])


def _segsum(x, rows2, cols2, zeros, zrow, with_hist):
    out_type = [jax.ShapeDtypeStruct((2, NPAD, D), _f32)]
    scratch = [
        pltpu.VMEM_SHARED((ACCR, D), _f32),
        pltpu.VMEM((PH, CH), _i32),
        pltpu.VMEM((PH, CH), _i32),
        pltpu.VMEM((CH, D), _f32),
        pltpu.VMEM((CH, D), _f32),
        pltpu.SemaphoreType.DMA,
        pltpu.SemaphoreType.DMA,
        pltpu.SemaphoreType.DMA,
        pltpu.SemaphoreType.DMA,
    ]
    if with_hist:
        out_type += [jax.ShapeDtypeStruct((NPAD,), _f32),
                     jax.ShapeDtypeStruct((NPAD,), _f32)]
        scratch = ([scratch[0], pltpu.VMEM_SHARED((ACCR,), _f32)]
                   + scratch[1:5] + [pltpu.VMEM((CH,), _f32)]
                   + scratch[5:]
                   + [pltpu.SemaphoreType.DMA, pltpu.SemaphoreType.DMA])
    res = pl.kernel(
        functools.partial(_segsum_body, with_hist),
        out_type=out_type,
        mesh=_mesh,
        scratch_types=scratch,
    )(x, rows2, cols2, zeros, zrow)
    return res if with_hist else res[0]


def _t2_body(a_ref, rs_ref, w_ref, o_ref):
    s = a_ref[0] + a_ref[1]
    z = jnp.dot(s, w_ref[...], preferred_element_type=_f32) / rs_ref[...]
    o_ref[...] = jnp.where(z > 0, z, jnp.exp(z) - 1.0)


def _t2(acc, rs, w, out_rows, blk):
    return pl.pallas_call(
        _t2_body,
        grid=(out_rows // blk,),
        in_specs=[
            pl.BlockSpec((2, blk, D), lambda i: (0, i, 0)),
            pl.BlockSpec((blk, 1), lambda i: (i, 0)),
            pl.BlockSpec((D, D), lambda i: (0, 0)),
        ],
        out_specs=pl.BlockSpec((blk, D), lambda i: (i, 0)),
        out_shape=jax.ShapeDtypeStruct((out_rows, D), _f32),
    )(acc, rs, w)


def kernel(node_feature, type_feature, length_feature, lane_feature,
           edge_index, struct_adj, struct_assign, fnc_assign, params):
    del struct_adj, struct_assign, fnc_assign

    nf = jnp.pad(node_feature.astype(_i32), (0, NPAD - N)).reshape(
        NPAD // EC, 1, EC)
    tf2 = jnp.pad(type_feature.astype(_i32), (0, NPAD - N)).reshape(NPAD, 1)
    lf2 = jnp.pad(length_feature.astype(_i32), (0, NPAD - N)).reshape(NPAD, 1)
    af2 = jnp.pad(lane_feature.astype(_i32), (0, NPAD - N)).reshape(NPAD, 1)
    pad_i = jnp.arange(EPAD, dtype=_i32)
    rows2 = jnp.concatenate(
        [edge_index[0].astype(_i32), N + pad_i % (ACCR - N)]
    ).reshape(NCHUNK, CH)
    cols2 = jnp.concatenate(
        [edge_index[1].astype(_i32), pad_i % 9973]
    ).reshape(NCHUNK, CH)

    z128 = jnp.zeros((NPAD, D), _f32)
    zrow = jnp.zeros((NPAD,), _f32)

    nt_p = jnp.pad(params["node_table"].astype(_f32), ((0, 0), (0, 64)))
    tt_p = jnp.pad(params["type_table"].astype(_f32), ((0, 12), (0, 0)))
    lt_p = jnp.pad(params["length_table"].astype(_f32), ((0, 28), (0, 0)))
    at_p = jnp.pad(params["lane_table"].astype(_f32), ((0, 6), (0, 0)))

    g = _embed(nf, nt_p)
    x1 = _t0(g, tf2, lf2, af2, tt_p, lt_p, at_p)

    ws = [p["gat_W"].astype(_f32) for p in params["layers"]]

    acc1, h0, h1 = _segsum(x1, rows2, cols2, z128, zrow, True)
    rs = (h0 + h1).reshape(NPAD, 1)
    x2 = _t2(acc1, rs, ws[0], NPAD, 2560)
    acc2 = _segsum(x2, rows2, cols2, z128, zrow, False)
    x3 = _t2(acc2, rs, ws[1], NPAD, 2560)
    acc3 = _segsum(x3, rows2, cols2, z128, zrow, False)
    return _t2(acc3, rs, ws[2], N, 2000)
